# baseline (speedup 1.0000x reference)
import functools

import jax
import jax.numpy as jnp
from jax import lax
from jax.experimental import pallas as pl
from jax.experimental.pallas import tpu as pltpu
from jax.experimental.pallas import tpu_sc as plsc

NC = 2
NS = 16
CHUNK = 128


def _sc_segment_sum(x, src3, dst3, zeros_hbm, n_pad, cpw):
    d = x.shape[1]
    rows_per_tile = n_pad // NS

    half = cpw // 2

    def full_body(x_hbm, src_hbm, dst_hbm, z_hbm, out_hbm, src_v, dst_v,
                  buf_a, buf_b, acc, sem_a, sem_b):
        c = lax.axis_index("c")
        s = lax.axis_index("s")
        wid = s * NC + c
        bufs = (buf_a, buf_b)
        sems = (sem_a, sem_b)

        for h in range(2):
            pltpu.sync_copy(src_hbm.at[wid, pl.ds(h * half, half)], src_v)
            pltpu.sync_copy(dst_hbm.at[wid, pl.ds(h * half, half)], dst_v)

            if h == 0:
                @pl.when(s == 0)
                def _():
                    pltpu.sync_copy(z_hbm, acc)

                plsc.subcore_barrier()

            @pl.loop(0, half)
            def _(j):
                pltpu.async_copy(x_hbm.at[src_v.at[j]], buf_a, sem_a).wait()
                pltpu.sync_copy(buf_a, acc.at[dst_v.at[j]], add=True)

        plsc.subcore_barrier()

        r0 = s * rows_per_tile
        pltpu.sync_copy(acc.at[pl.ds(r0, rows_per_tile)],
                        out_hbm.at[c, pl.ds(r0, rows_per_tile)])

    mesh = plsc.VectorSubcoreMesh(core_axis_name="c", subcore_axis_name="s")
    k = pl.kernel(
        full_body,
        out_type=jax.ShapeDtypeStruct((NC, n_pad, d), jnp.float32),
        mesh=mesh,
        scratch_types=[
            pltpu.VMEM((half, CHUNK), jnp.int32),
            pltpu.VMEM((half, CHUNK), jnp.int32),
            pltpu.VMEM((CHUNK, d), jnp.float32),
            pltpu.VMEM((CHUNK, d), jnp.float32),
            pltpu.VMEM_SHARED((n_pad, d), jnp.float32),
            pltpu.SemaphoreType.DMA,
            pltpu.SemaphoreType.DMA,
        ],
    )
    return k(x, src3, dst3, zeros_hbm)


def _mm(a, b_t):
    return lax.dot_general(a, b_t, (((1,), (1,)), ((), ())),
                           preferred_element_type=jnp.float32,
                           precision=lax.Precision.HIGHEST)


def _accum_stats(h, sum_out, sq_out, acc_sum, acc_sq, nb):
    j = pl.program_id(0)
    s = jnp.sum(h, axis=0, keepdims=True)
    q = jnp.sum(h * h, axis=0, keepdims=True)

    @pl.when(j == 0)
    def _():
        acc_sum[...] = s
        acc_sq[...] = q

    @pl.when(j > 0)
    def _():
        acc_sum[...] += s
        acc_sq[...] += q

    @pl.when(j == nb - 1)
    def _():
        sum_out[...] = acc_sum[...]
        sq_out[...] = acc_sq[...]


def _l1_body(x_ref, p_ref, w1a_ref, w1b_ref, b1_ref, h_out, sum_out, sq_out,
             acc_sum, acc_sq, *, nb):
    agg = p_ref[0] + p_ref[1]
    h = _mm(x_ref[...], w1a_ref[...]) + _mm(agg, w1b_ref[...]) + b1_ref[...]
    h_out[...] = h
    _accum_stats(h, sum_out, sq_out, acc_sum, acc_sq, nb)


def _l2_body(h_ref, sum_ref, sq_ref, g1_ref, be1_ref, w2_ref, b2_ref,
             h2_out, sum_out, sq_out, acc_sum, acc_sq, *, n, nb, eps):
    mean = sum_ref[...] * (1.0 / n)
    var = sq_ref[...] * (1.0 / n) - mean * mean
    h = (h_ref[...] - mean) * lax.rsqrt(var + eps) * g1_ref[...] + be1_ref[...]
    h = jnp.maximum(h, 0.0)
    h2 = _mm(h, w2_ref[...]) + b2_ref[...]
    h2_out[...] = h2
    _accum_stats(h2, sum_out, sq_out, acc_sum, acc_sq, nb)


def _l3_body(h2_ref, sum_ref, sq_ref, g2_ref, be2_ref, o_ref, *, n, eps):
    mean = sum_ref[...] * (1.0 / n)
    var = sq_ref[...] * (1.0 / n) - mean * mean
    h2 = (h2_ref[...] - mean) * lax.rsqrt(var + eps) * g2_ref[...] \
        + be2_ref[...]
    o_ref[...] = jnp.maximum(h2, 0.0)


def _mlp(x, partials, W1, b1, g1, be1, W2, b2, g2, be2, eps):
    n, d = x.shape
    d_hid = W1.shape[0]
    d_out = W2.shape[0]
    bl = 1000
    nb = n // bl
    w1a = W1[:, :d]
    w1b = W1[:, d:]
    row = lambda j: (j, 0)
    fixed = lambda j: (0, 0)
    stat_spec = lambda w: pl.BlockSpec((1, w), fixed)
    vec_spec = lambda w: pl.BlockSpec((w,), lambda j: (0,))

    h1pre, sum1, sq1 = pl.pallas_call(
        functools.partial(_l1_body, nb=nb),
        grid=(nb,),
        in_specs=[
            pl.BlockSpec((bl, d), row),
            pl.BlockSpec((2, bl, d), lambda j: (0, j, 0)),
            pl.BlockSpec((d_hid, d), fixed),
            pl.BlockSpec((d_hid, d), fixed),
            vec_spec(d_hid),
        ],
        out_specs=[pl.BlockSpec((bl, d_hid), row), stat_spec(d_hid),
                   stat_spec(d_hid)],
        out_shape=[jax.ShapeDtypeStruct((n, d_hid), jnp.float32),
                   jax.ShapeDtypeStruct((1, d_hid), jnp.float32),
                   jax.ShapeDtypeStruct((1, d_hid), jnp.float32)],
        scratch_shapes=[pltpu.VMEM((1, d_hid), jnp.float32),
                        pltpu.VMEM((1, d_hid), jnp.float32)],
    )(x, partials, w1a, w1b, b1)

    h2pre, sum2, sq2 = pl.pallas_call(
        functools.partial(_l2_body, n=n, nb=nb, eps=eps),
        grid=(nb,),
        in_specs=[
            pl.BlockSpec((bl, d_hid), row),
            stat_spec(d_hid),
            stat_spec(d_hid),
            vec_spec(d_hid),
            vec_spec(d_hid),
            pl.BlockSpec((d_out, d_hid), fixed),
            vec_spec(d_out),
        ],
        out_specs=[pl.BlockSpec((bl, d_out), row), stat_spec(d_out),
                   stat_spec(d_out)],
        out_shape=[jax.ShapeDtypeStruct((n, d_out), jnp.float32),
                   jax.ShapeDtypeStruct((1, d_out), jnp.float32),
                   jax.ShapeDtypeStruct((1, d_out), jnp.float32)],
        scratch_shapes=[pltpu.VMEM((1, d_out), jnp.float32),
                        pltpu.VMEM((1, d_out), jnp.float32)],
    )(h1pre, sum1, sq1, g1, be1, W2, b2)

    out = pl.pallas_call(
        functools.partial(_l3_body, n=n, eps=eps),
        grid=(nb,),
        in_specs=[
            pl.BlockSpec((bl, d_out), row),
            stat_spec(d_out),
            stat_spec(d_out),
            vec_spec(d_out),
            vec_spec(d_out),
        ],
        out_specs=pl.BlockSpec((bl, d_out), row),
        out_shape=jax.ShapeDtypeStruct((n, d_out), jnp.float32),
    )(h2pre, sum2, sq2, g2, be2)
    return out


def kernel(x, edge_index, W1, b1, g1, be1, W2, b2, g2, be2):
    n, d = x.shape
    e = edge_index.shape[1]
    eps = 1e-5

    src = edge_index[0].astype(jnp.int32)
    dst = edge_index[1].astype(jnp.int32)
    nw = NC * NS
    cpw = -(-e // (nw * CHUNK))
    cpw = -(-cpw // 4) * 4
    e_pad = nw * cpw * CHUNK
    n_pad = -(-(n + 1) // (NS * 8)) * (NS * 8)
    pad = e_pad - e
    src = jnp.concatenate([src, jnp.zeros((pad,), jnp.int32)])
    dst = jnp.concatenate([dst, jnp.full((pad,), n, jnp.int32)])
    src3 = src.reshape(nw, cpw, CHUNK)
    dst3 = dst.reshape(nw, cpw, CHUNK)
    zeros_hbm = jnp.zeros((n_pad, d), jnp.float32)

    partials = _sc_segment_sum(x, src3, dst3, zeros_hbm, n_pad, cpw)
    return _mlp(x, partials, W1, b1, g1, be1, W2, b2, g2, be2, eps)


# speedup vs baseline: 1.0004x; 1.0004x over previous
import functools

import jax
import jax.numpy as jnp
from jax import lax
from jax.experimental import pallas as pl
from jax.experimental.pallas import tpu as pltpu
from jax.experimental.pallas import tpu_sc as plsc

NC = 2
NS = 16
CHUNK = 128


def _sc_segment_sum(x, src3, dst3, zeros_hbm, n_pad, cpw):
    d = x.shape[1]
    rows_per_tile = n_pad // NS

    half = cpw // 2

    def full_body(x_hbm, src_hbm, dst_hbm, z_hbm, out_hbm, src_v, dst_v,
                  buf_a, buf_b, acc, sem_a, sem_b):
        c = lax.axis_index("c")
        s = lax.axis_index("s")
        wid = s * NC + c
        bufs = (buf_a, buf_b)
        sems = (sem_a, sem_b)

        for h in range(2):
            pltpu.sync_copy(src_hbm.at[wid, pl.ds(h * half, half)], src_v)
            pltpu.sync_copy(dst_hbm.at[wid, pl.ds(h * half, half)], dst_v)

            if h == 0:
                @pl.when(s == 0)
                def _():
                    pltpu.sync_copy(z_hbm, acc)

                plsc.subcore_barrier()

            def step(j, carry):
                pltpu.async_copy(x_hbm.at[src_v.at[j]], buf_a, sem_a).wait()
                pltpu.sync_copy(buf_a, acc.at[dst_v.at[j]], add=True)
                return carry

            lax.fori_loop(0, half, step, 0)

        plsc.subcore_barrier()

        r0 = s * rows_per_tile
        pltpu.sync_copy(acc.at[pl.ds(r0, rows_per_tile)],
                        out_hbm.at[c, pl.ds(r0, rows_per_tile)])

    mesh = plsc.VectorSubcoreMesh(core_axis_name="c", subcore_axis_name="s")
    k = pl.kernel(
        full_body,
        out_type=jax.ShapeDtypeStruct((NC, n_pad, d), jnp.float32),
        mesh=mesh,
        scratch_types=[
            pltpu.VMEM((half, CHUNK), jnp.int32),
            pltpu.VMEM((half, CHUNK), jnp.int32),
            pltpu.VMEM((CHUNK, d), jnp.float32),
            pltpu.VMEM((CHUNK, d), jnp.float32),
            pltpu.VMEM_SHARED((n_pad, d), jnp.float32),
            pltpu.SemaphoreType.DMA,
            pltpu.SemaphoreType.DMA,
        ],
    )
    return k(x, src3, dst3, zeros_hbm)


def _mm(a, b_t):
    return lax.dot_general(a, b_t, (((1,), (1,)), ((), ())),
                           preferred_element_type=jnp.float32,
                           precision=lax.Precision.HIGHEST)


def _accum_stats(h, sum_out, sq_out, acc_sum, acc_sq, nb):
    j = pl.program_id(0)
    s = jnp.sum(h, axis=0, keepdims=True)
    q = jnp.sum(h * h, axis=0, keepdims=True)

    @pl.when(j == 0)
    def _():
        acc_sum[...] = s
        acc_sq[...] = q

    @pl.when(j > 0)
    def _():
        acc_sum[...] += s
        acc_sq[...] += q

    @pl.when(j == nb - 1)
    def _():
        sum_out[...] = acc_sum[...]
        sq_out[...] = acc_sq[...]


def _l1_body(x_ref, p_ref, w1a_ref, w1b_ref, b1_ref, h_out, sum_out, sq_out,
             acc_sum, acc_sq, *, nb):
    agg = p_ref[0] + p_ref[1]
    h = _mm(x_ref[...], w1a_ref[...]) + _mm(agg, w1b_ref[...]) + b1_ref[...]
    h_out[...] = h
    _accum_stats(h, sum_out, sq_out, acc_sum, acc_sq, nb)


def _l2_body(h_ref, sum_ref, sq_ref, g1_ref, be1_ref, w2_ref, b2_ref,
             h2_out, sum_out, sq_out, acc_sum, acc_sq, *, n, nb, eps):
    mean = sum_ref[...] * (1.0 / n)
    var = sq_ref[...] * (1.0 / n) - mean * mean
    h = (h_ref[...] - mean) * lax.rsqrt(var + eps) * g1_ref[...] + be1_ref[...]
    h = jnp.maximum(h, 0.0)
    h2 = _mm(h, w2_ref[...]) + b2_ref[...]
    h2_out[...] = h2
    _accum_stats(h2, sum_out, sq_out, acc_sum, acc_sq, nb)


def _l3_body(h2_ref, sum_ref, sq_ref, g2_ref, be2_ref, o_ref, *, n, eps):
    mean = sum_ref[...] * (1.0 / n)
    var = sq_ref[...] * (1.0 / n) - mean * mean
    h2 = (h2_ref[...] - mean) * lax.rsqrt(var + eps) * g2_ref[...] \
        + be2_ref[...]
    o_ref[...] = jnp.maximum(h2, 0.0)


def _mlp(x, partials, W1, b1, g1, be1, W2, b2, g2, be2, eps):
    n, d = x.shape
    d_hid = W1.shape[0]
    d_out = W2.shape[0]
    bl = 1000
    nb = n // bl
    w1a = W1[:, :d]
    w1b = W1[:, d:]
    row = lambda j: (j, 0)
    fixed = lambda j: (0, 0)
    stat_spec = lambda w: pl.BlockSpec((1, w), fixed)
    vec_spec = lambda w: pl.BlockSpec((w,), lambda j: (0,))

    h1pre, sum1, sq1 = pl.pallas_call(
        functools.partial(_l1_body, nb=nb),
        grid=(nb,),
        in_specs=[
            pl.BlockSpec((bl, d), row),
            pl.BlockSpec((2, bl, d), lambda j: (0, j, 0)),
            pl.BlockSpec((d_hid, d), fixed),
            pl.BlockSpec((d_hid, d), fixed),
            vec_spec(d_hid),
        ],
        out_specs=[pl.BlockSpec((bl, d_hid), row), stat_spec(d_hid),
                   stat_spec(d_hid)],
        out_shape=[jax.ShapeDtypeStruct((n, d_hid), jnp.float32),
                   jax.ShapeDtypeStruct((1, d_hid), jnp.float32),
                   jax.ShapeDtypeStruct((1, d_hid), jnp.float32)],
        scratch_shapes=[pltpu.VMEM((1, d_hid), jnp.float32),
                        pltpu.VMEM((1, d_hid), jnp.float32)],
    )(x, partials, w1a, w1b, b1)

    h2pre, sum2, sq2 = pl.pallas_call(
        functools.partial(_l2_body, n=n, nb=nb, eps=eps),
        grid=(nb,),
        in_specs=[
            pl.BlockSpec((bl, d_hid), row),
            stat_spec(d_hid),
            stat_spec(d_hid),
            vec_spec(d_hid),
            vec_spec(d_hid),
            pl.BlockSpec((d_out, d_hid), fixed),
            vec_spec(d_out),
        ],
        out_specs=[pl.BlockSpec((bl, d_out), row), stat_spec(d_out),
                   stat_spec(d_out)],
        out_shape=[jax.ShapeDtypeStruct((n, d_out), jnp.float32),
                   jax.ShapeDtypeStruct((1, d_out), jnp.float32),
                   jax.ShapeDtypeStruct((1, d_out), jnp.float32)],
        scratch_shapes=[pltpu.VMEM((1, d_out), jnp.float32),
                        pltpu.VMEM((1, d_out), jnp.float32)],
    )(h1pre, sum1, sq1, g1, be1, W2, b2)

    out = pl.pallas_call(
        functools.partial(_l3_body, n=n, eps=eps),
        grid=(nb,),
        in_specs=[
            pl.BlockSpec((bl, d_out), row),
            stat_spec(d_out),
            stat_spec(d_out),
            vec_spec(d_out),
            vec_spec(d_out),
        ],
        out_specs=pl.BlockSpec((bl, d_out), row),
        out_shape=jax.ShapeDtypeStruct((n, d_out), jnp.float32),
    )(h2pre, sum2, sq2, g2, be2)
    return out


def kernel(x, edge_index, W1, b1, g1, be1, W2, b2, g2, be2):
    n, d = x.shape
    e = edge_index.shape[1]
    eps = 1e-5

    src = edge_index[0].astype(jnp.int32)
    dst = edge_index[1].astype(jnp.int32)
    nw = NC * NS
    cpw = -(-e // (nw * CHUNK))
    cpw = -(-cpw // 4) * 4
    e_pad = nw * cpw * CHUNK
    n_pad = -(-(n + 1) // (NS * 8)) * (NS * 8)
    pad = e_pad - e
    src = jnp.concatenate([src, jnp.zeros((pad,), jnp.int32)])
    dst = jnp.concatenate([dst, jnp.full((pad,), n, jnp.int32)])
    src3 = src.reshape(nw, cpw, CHUNK)
    dst3 = dst.reshape(nw, cpw, CHUNK)
    zeros_hbm = jnp.zeros((n_pad, d), jnp.float32)

    partials = _sc_segment_sum(x, src3, dst3, zeros_hbm, n_pad, cpw)
    return _mlp(x, partials, W1, b1, g1, be1, W2, b2, g2, be2, eps)


# speedup vs baseline: 1.4423x; 1.4418x over previous
import functools

import jax
import jax.numpy as jnp
from jax import lax
from jax.experimental import pallas as pl
from jax.experimental.pallas import tpu as pltpu
from jax.experimental.pallas import tpu_sc as plsc

NC = 2
NS = 16
CHUNK = 128


def _sc_segment_sum(x, src3, dst3, zeros_hbm, n_pad, cpw):
    d = x.shape[1]
    rows_per_tile = n_pad // NS

    def full_body(x_hbm, src_hbm, dst_hbm, z_hbm, out_hbm, src_v, dst_v,
                  buf_a, acc, sem_a):
        c = lax.axis_index("c")
        s = lax.axis_index("s")
        wid = s * NC + c

        pltpu.sync_copy(src_hbm.at[wid], src_v)
        pltpu.sync_copy(dst_hbm.at[wid], dst_v)

        @pl.when(s == 0)
        def _():
            pltpu.sync_copy(z_hbm, acc)

        plsc.subcore_barrier()

        def step(j, carry):
            pltpu.async_copy(x_hbm.at[src_v.at[j]], buf_a, sem_a).wait()
            pltpu.sync_copy(buf_a, acc.at[dst_v.at[j]], add=True)
            return carry

        lax.fori_loop(0, cpw, step, 0)
        plsc.subcore_barrier()

        r0 = s * rows_per_tile
        pltpu.sync_copy(acc.at[pl.ds(r0, rows_per_tile)],
                        out_hbm.at[c, pl.ds(r0, rows_per_tile)])

    mesh = plsc.VectorSubcoreMesh(core_axis_name="c", subcore_axis_name="s")
    k = pl.kernel(
        full_body,
        out_type=jax.ShapeDtypeStruct((NC, n_pad, d), jnp.float32),
        mesh=mesh,
        scratch_types=[
            pltpu.VMEM((cpw, CHUNK), jnp.int32),
            pltpu.VMEM((cpw, CHUNK), jnp.int32),
            pltpu.VMEM((CHUNK, d), jnp.float32),
            pltpu.VMEM_SHARED((n_pad, d), jnp.float32),
            pltpu.SemaphoreType.DMA,
        ],
    )
    return k(x, src3, dst3, zeros_hbm)


def _mm(a, b_t):
    return lax.dot_general(a, b_t, (((1,), (1,)), ((), ())),
                           preferred_element_type=jnp.float32,
                           precision=lax.Precision.HIGHEST)


def _accum_stats(h, sum_out, sq_out, acc_sum, acc_sq, nb):
    j = pl.program_id(0)
    s = jnp.sum(h, axis=0, keepdims=True)
    q = jnp.sum(h * h, axis=0, keepdims=True)

    @pl.when(j == 0)
    def _():
        acc_sum[...] = s
        acc_sq[...] = q

    @pl.when(j > 0)
    def _():
        acc_sum[...] += s
        acc_sq[...] += q

    @pl.when(j == nb - 1)
    def _():
        sum_out[...] = acc_sum[...]
        sq_out[...] = acc_sq[...]


def _l1_body(x_ref, p_ref, w1a_ref, w1b_ref, b1_ref, h_out, sum_out, sq_out,
             acc_sum, acc_sq, *, nb):
    agg = p_ref[0] + p_ref[1]
    h = _mm(x_ref[...], w1a_ref[...]) + _mm(agg, w1b_ref[...]) + b1_ref[...]
    h_out[...] = h
    _accum_stats(h, sum_out, sq_out, acc_sum, acc_sq, nb)


def _l2_body(h_ref, sum_ref, sq_ref, g1_ref, be1_ref, w2_ref, b2_ref,
             h2_out, sum_out, sq_out, acc_sum, acc_sq, *, n, nb, eps):
    mean = sum_ref[...] * (1.0 / n)
    var = sq_ref[...] * (1.0 / n) - mean * mean
    h = (h_ref[...] - mean) * lax.rsqrt(var + eps) * g1_ref[...] + be1_ref[...]
    h = jnp.maximum(h, 0.0)
    h2 = _mm(h, w2_ref[...]) + b2_ref[...]
    h2_out[...] = h2
    _accum_stats(h2, sum_out, sq_out, acc_sum, acc_sq, nb)


def _l3_body(h2_ref, sum_ref, sq_ref, g2_ref, be2_ref, o_ref, *, n, eps):
    mean = sum_ref[...] * (1.0 / n)
    var = sq_ref[...] * (1.0 / n) - mean * mean
    h2 = (h2_ref[...] - mean) * lax.rsqrt(var + eps) * g2_ref[...] \
        + be2_ref[...]
    o_ref[...] = jnp.maximum(h2, 0.0)


def _mlp(x, partials, W1, b1, g1, be1, W2, b2, g2, be2, eps):
    n, d = x.shape
    d_hid = W1.shape[0]
    d_out = W2.shape[0]
    bl = 1000
    nb = n // bl
    w1a = W1[:, :d]
    w1b = W1[:, d:]
    row = lambda j: (j, 0)
    fixed = lambda j: (0, 0)
    stat_spec = lambda w: pl.BlockSpec((1, w), fixed)
    vec_spec = lambda w: pl.BlockSpec((w,), lambda j: (0,))

    h1pre, sum1, sq1 = pl.pallas_call(
        functools.partial(_l1_body, nb=nb),
        grid=(nb,),
        in_specs=[
            pl.BlockSpec((bl, d), row),
            pl.BlockSpec((2, bl, d), lambda j: (0, j, 0)),
            pl.BlockSpec((d_hid, d), fixed),
            pl.BlockSpec((d_hid, d), fixed),
            vec_spec(d_hid),
        ],
        out_specs=[pl.BlockSpec((bl, d_hid), row), stat_spec(d_hid),
                   stat_spec(d_hid)],
        out_shape=[jax.ShapeDtypeStruct((n, d_hid), jnp.float32),
                   jax.ShapeDtypeStruct((1, d_hid), jnp.float32),
                   jax.ShapeDtypeStruct((1, d_hid), jnp.float32)],
        scratch_shapes=[pltpu.VMEM((1, d_hid), jnp.float32),
                        pltpu.VMEM((1, d_hid), jnp.float32)],
    )(x, partials, w1a, w1b, b1)

    h2pre, sum2, sq2 = pl.pallas_call(
        functools.partial(_l2_body, n=n, nb=nb, eps=eps),
        grid=(nb,),
        in_specs=[
            pl.BlockSpec((bl, d_hid), row),
            stat_spec(d_hid),
            stat_spec(d_hid),
            vec_spec(d_hid),
            vec_spec(d_hid),
            pl.BlockSpec((d_out, d_hid), fixed),
            vec_spec(d_out),
        ],
        out_specs=[pl.BlockSpec((bl, d_out), row), stat_spec(d_out),
                   stat_spec(d_out)],
        out_shape=[jax.ShapeDtypeStruct((n, d_out), jnp.float32),
                   jax.ShapeDtypeStruct((1, d_out), jnp.float32),
                   jax.ShapeDtypeStruct((1, d_out), jnp.float32)],
        scratch_shapes=[pltpu.VMEM((1, d_out), jnp.float32),
                        pltpu.VMEM((1, d_out), jnp.float32)],
    )(h1pre, sum1, sq1, g1, be1, W2, b2)

    out = pl.pallas_call(
        functools.partial(_l3_body, n=n, eps=eps),
        grid=(nb,),
        in_specs=[
            pl.BlockSpec((bl, d_out), row),
            stat_spec(d_out),
            stat_spec(d_out),
            vec_spec(d_out),
            vec_spec(d_out),
        ],
        out_specs=pl.BlockSpec((bl, d_out), row),
        out_shape=jax.ShapeDtypeStruct((n, d_out), jnp.float32),
    )(h2pre, sum2, sq2, g2, be2)
    return out


def kernel(x, edge_index, W1, b1, g1, be1, W2, b2, g2, be2):
    n, d = x.shape
    e = edge_index.shape[1]
    eps = 1e-5

    src = edge_index[0].astype(jnp.int32)
    dst = edge_index[1].astype(jnp.int32)
    nw = NC * NS
    cpw = -(-e // (nw * CHUNK))
    e_pad = nw * cpw * CHUNK
    n_pad = -(-(n + 1) // (NS * 8)) * (NS * 8)
    pad = e_pad - e
    src = jnp.concatenate([src, jnp.zeros((pad,), jnp.int32)])
    dst = jnp.concatenate([dst, jnp.full((pad,), n, jnp.int32)])
    src3 = src.reshape(nw, cpw, CHUNK)
    dst3 = dst.reshape(nw, cpw, CHUNK)
    zeros_hbm = jnp.zeros((n_pad, d), jnp.float32)

    partials = _sc_segment_sum(x, src3, dst3, zeros_hbm, n_pad, cpw)
    return _mlp(x, partials, W1, b1, g1, be1, W2, b2, g2, be2, eps)


# speedup vs baseline: 2.2604x; 1.5672x over previous
import functools

import jax
import jax.numpy as jnp
from jax import lax
from jax.experimental import pallas as pl
from jax.experimental.pallas import tpu as pltpu
from jax.experimental.pallas import tpu_sc as plsc

NC = 2
NS = 16
CHUNK = 128


def _sc_segment_sum(x, src3, dst3, zeros_hbm, n_pad, cpw):
    d = x.shape[1]
    rows_per_tile = n_pad // NS

    def full_body(x_hbm, src_hbm, dst_hbm, z_hbm, out_hbm, src_v, dst_v,
                  buf_a, acc, sem_a):
        c = lax.axis_index("c")
        s = lax.axis_index("s")
        wid = s * NC + c

        pltpu.sync_copy(src_hbm.at[wid], src_v)
        pltpu.sync_copy(dst_hbm.at[wid], dst_v)

        @pl.when(s == 0)
        def _():
            pltpu.sync_copy(z_hbm, acc)

        plsc.subcore_barrier()

        def step(j, carry):
            pltpu.async_copy(x_hbm.at[src_v.at[j]], buf_a, sem_a).wait()
            pltpu.sync_copy(buf_a, acc.at[dst_v.at[j]], add=True)
            return carry

        lax.fori_loop(0, cpw, step, 0)
        plsc.subcore_barrier()

        r0 = s * rows_per_tile
        pltpu.sync_copy(acc.at[pl.ds(r0, rows_per_tile)],
                        out_hbm.at[c, pl.ds(r0, rows_per_tile)])

    mesh = plsc.VectorSubcoreMesh(core_axis_name="c", subcore_axis_name="s")
    k = pl.kernel(
        full_body,
        out_type=jax.ShapeDtypeStruct((NC, n_pad, d), jnp.float32),
        mesh=mesh,
        scratch_types=[
            pltpu.VMEM((cpw, CHUNK), jnp.int32),
            pltpu.VMEM((cpw, CHUNK), jnp.int32),
            pltpu.VMEM((CHUNK, d), jnp.float32),
            pltpu.VMEM_SHARED((n_pad, d), jnp.float32),
            pltpu.SemaphoreType.DMA,
        ],
    )
    return k(x, src3, dst3, zeros_hbm)


def _mm(a, b_t):
    return lax.dot_general(a, b_t, (((1,), (1,)), ((), ())),
                           preferred_element_type=jnp.float32,
                           precision=lax.Precision.HIGHEST)


def _accum_stats(h, sum_out, sq_out, acc_sum, acc_sq, nb):
    j = pl.program_id(0)
    s = jnp.sum(h, axis=0, keepdims=True)
    q = jnp.sum(h * h, axis=0, keepdims=True)

    @pl.when(j == 0)
    def _():
        acc_sum[...] = s
        acc_sq[...] = q

    @pl.when(j > 0)
    def _():
        acc_sum[...] += s
        acc_sq[...] += q

    @pl.when(j == nb - 1)
    def _():
        sum_out[...] = acc_sum[...]
        sq_out[...] = acc_sq[...]


def _l1_body(x_ref, p_ref, w1a_ref, w1b_ref, b1_ref, h_out, sum_out, sq_out,
             acc_sum, acc_sq, *, nb):
    agg = p_ref[0] + p_ref[1]
    h = _mm(x_ref[...], w1a_ref[...]) + _mm(agg, w1b_ref[...]) + b1_ref[...]
    h_out[...] = h
    _accum_stats(h, sum_out, sq_out, acc_sum, acc_sq, nb)


def _l2_body(h_ref, sum_ref, sq_ref, g1_ref, be1_ref, w2_ref, b2_ref,
             h2_out, sum_out, sq_out, acc_sum, acc_sq, *, n, nb, eps):
    mean = sum_ref[...] * (1.0 / n)
    var = sq_ref[...] * (1.0 / n) - mean * mean
    h = (h_ref[...] - mean) * lax.rsqrt(var + eps) * g1_ref[...] + be1_ref[...]
    h = jnp.maximum(h, 0.0)
    h2 = _mm(h, w2_ref[...]) + b2_ref[...]
    h2_out[...] = h2
    _accum_stats(h2, sum_out, sq_out, acc_sum, acc_sq, nb)


def _l3_body(h2_ref, sum_ref, sq_ref, g2_ref, be2_ref, o_ref, *, n, eps):
    mean = sum_ref[...] * (1.0 / n)
    var = sq_ref[...] * (1.0 / n) - mean * mean
    h2 = (h2_ref[...] - mean) * lax.rsqrt(var + eps) * g2_ref[...] \
        + be2_ref[...]
    o_ref[...] = jnp.maximum(h2, 0.0)


def _mlp(x, partials, W1, b1, g1, be1, W2, b2, g2, be2, eps):
    n, d = x.shape
    d_hid = W1.shape[0]
    d_out = W2.shape[0]
    bl = 1000
    nb = n // bl
    w1a = W1[:, :d]
    w1b = W1[:, d:]
    row = lambda j: (j, 0)
    fixed = lambda j: (0, 0)
    stat_spec = lambda w: pl.BlockSpec((1, w), fixed)
    vec_spec = lambda w: pl.BlockSpec((w,), lambda j: (0,))

    h1pre, sum1, sq1 = pl.pallas_call(
        functools.partial(_l1_body, nb=nb),
        grid=(nb,),
        in_specs=[
            pl.BlockSpec((bl, d), row),
            pl.BlockSpec((2, bl, d), lambda j: (0, j, 0)),
            pl.BlockSpec((d_hid, d), fixed),
            pl.BlockSpec((d_hid, d), fixed),
            vec_spec(d_hid),
        ],
        out_specs=[pl.BlockSpec((bl, d_hid), row), stat_spec(d_hid),
                   stat_spec(d_hid)],
        out_shape=[jax.ShapeDtypeStruct((n, d_hid), jnp.float32),
                   jax.ShapeDtypeStruct((1, d_hid), jnp.float32),
                   jax.ShapeDtypeStruct((1, d_hid), jnp.float32)],
        scratch_shapes=[pltpu.VMEM((1, d_hid), jnp.float32),
                        pltpu.VMEM((1, d_hid), jnp.float32)],
    )(x, partials, w1a, w1b, b1)

    h2pre, sum2, sq2 = pl.pallas_call(
        functools.partial(_l2_body, n=n, nb=nb, eps=eps),
        grid=(nb,),
        in_specs=[
            pl.BlockSpec((bl, d_hid), row),
            stat_spec(d_hid),
            stat_spec(d_hid),
            vec_spec(d_hid),
            vec_spec(d_hid),
            pl.BlockSpec((d_out, d_hid), fixed),
            vec_spec(d_out),
        ],
        out_specs=[pl.BlockSpec((bl, d_out), row), stat_spec(d_out),
                   stat_spec(d_out)],
        out_shape=[jax.ShapeDtypeStruct((n, d_out), jnp.float32),
                   jax.ShapeDtypeStruct((1, d_out), jnp.float32),
                   jax.ShapeDtypeStruct((1, d_out), jnp.float32)],
        scratch_shapes=[pltpu.VMEM((1, d_out), jnp.float32),
                        pltpu.VMEM((1, d_out), jnp.float32)],
    )(h1pre, sum1, sq1, g1, be1, W2, b2)

    out = pl.pallas_call(
        functools.partial(_l3_body, n=n, eps=eps),
        grid=(nb,),
        in_specs=[
            pl.BlockSpec((bl, d_out), row),
            stat_spec(d_out),
            stat_spec(d_out),
            vec_spec(d_out),
            vec_spec(d_out),
        ],
        out_specs=pl.BlockSpec((bl, d_out), row),
        out_shape=jax.ShapeDtypeStruct((n, d_out), jnp.float32),
    )(h2pre, sum2, sq2, g2, be2)
    return out


def kernel(x, edge_index, W1, b1, g1, be1, W2, b2, g2, be2):
    n, d = x.shape
    e = edge_index.shape[1]
    eps = 1e-5

    src = edge_index[0].astype(jnp.int32)
    dst = edge_index[1].astype(jnp.int32)
    nw = NC * NS
    cpw = -(-e // (nw * CHUNK))
    e_pad = nw * cpw * CHUNK
    n_pad = -(-(n + 1) // (NS * 8)) * (NS * 8)
    pad = e_pad - e
    pad_i = jnp.arange(pad, dtype=jnp.int32)
    src = jnp.concatenate([src, pad_i % n])
    dst = jnp.concatenate([dst, n + pad_i % (n_pad - n)])
    src3 = src.reshape(nw, cpw, CHUNK)
    dst3 = dst.reshape(nw, cpw, CHUNK)
    zeros_hbm = jnp.zeros((n_pad, d), jnp.float32)

    partials = _sc_segment_sum(x, src3, dst3, zeros_hbm, n_pad, cpw)
    return _mlp(x, partials, W1, b1, g1, be1, W2, b2, g2, be2, eps)


# speedup vs baseline: 2.9459x; 1.3033x over previous
import functools

import jax
import jax.numpy as jnp
from jax import lax
from jax.experimental import pallas as pl
from jax.experimental.pallas import tpu as pltpu
from jax.experimental.pallas import tpu_sc as plsc

NC = 2
NS = 16
CHUNK = 128


def _sc_segment_sum(x, src3, dst3, zeros_hbm, n_pad, cpw):
    d = x.shape[1]
    rows_per_tile = n_pad // NS

    half = cpw // 2

    def full_body(x_hbm, src_hbm, dst_hbm, z_hbm, out_hbm, src_v, dst_v,
                  buf_a, buf_b, acc, sem_a, sem_b):
        c = lax.axis_index("c")
        s = lax.axis_index("s")
        wid = s * NC + c
        bufs = (buf_a, buf_b)
        sems = (sem_a, sem_b)

        for h in range(2):
            pltpu.sync_copy(src_hbm.at[wid, pl.ds(h * half, half)], src_v)
            pltpu.sync_copy(dst_hbm.at[wid, pl.ds(h * half, half)], dst_v)

            if h == 0:
                @pl.when(s == 0)
                def _():
                    pltpu.sync_copy(z_hbm, acc)

                plsc.subcore_barrier()

            for b in range(2):
                pltpu.async_copy(x_hbm.at[src_v.at[b]], bufs[b], sems[b])

            def ring(t, carry):
                g = 2 * t
                for b in range(2):
                    j = g + b
                    pltpu.make_async_copy(x_hbm.at[src_v.at[j]], bufs[b],
                                          sems[b]).wait()
                    pltpu.sync_copy(bufs[b], acc.at[dst_v.at[j]], add=True)
                    pltpu.async_copy(x_hbm.at[src_v.at[j + 2]], bufs[b],
                                     sems[b])
                return carry

            lax.fori_loop(0, (half - 2) // 2, ring, 0)

            for b in range(2):
                j = half - 2 + b
                pltpu.make_async_copy(x_hbm.at[src_v.at[j]], bufs[b],
                                      sems[b]).wait()
                pltpu.sync_copy(bufs[b], acc.at[dst_v.at[j]], add=True)

        plsc.subcore_barrier()

        r0 = s * rows_per_tile
        pltpu.sync_copy(acc.at[pl.ds(r0, rows_per_tile)],
                        out_hbm.at[c, pl.ds(r0, rows_per_tile)])

    mesh = plsc.VectorSubcoreMesh(core_axis_name="c", subcore_axis_name="s")
    k = pl.kernel(
        full_body,
        out_type=jax.ShapeDtypeStruct((NC, n_pad, d), jnp.float32),
        mesh=mesh,
        scratch_types=[
            pltpu.VMEM((half, CHUNK), jnp.int32),
            pltpu.VMEM((half, CHUNK), jnp.int32),
            pltpu.VMEM((CHUNK, d), jnp.float32),
            pltpu.VMEM((CHUNK, d), jnp.float32),
            pltpu.VMEM_SHARED((n_pad, d), jnp.float32),
            pltpu.SemaphoreType.DMA,
            pltpu.SemaphoreType.DMA,
        ],
    )
    return k(x, src3, dst3, zeros_hbm)


def _mm(a, b_t):
    return lax.dot_general(a, b_t, (((1,), (1,)), ((), ())),
                           preferred_element_type=jnp.float32,
                           precision=lax.Precision.HIGHEST)


def _accum_stats(h, sum_out, sq_out, acc_sum, acc_sq, nb):
    j = pl.program_id(0)
    s = jnp.sum(h, axis=0, keepdims=True)
    q = jnp.sum(h * h, axis=0, keepdims=True)

    @pl.when(j == 0)
    def _():
        acc_sum[...] = s
        acc_sq[...] = q

    @pl.when(j > 0)
    def _():
        acc_sum[...] += s
        acc_sq[...] += q

    @pl.when(j == nb - 1)
    def _():
        sum_out[...] = acc_sum[...]
        sq_out[...] = acc_sq[...]


def _l1_body(x_ref, p_ref, w1a_ref, w1b_ref, b1_ref, h_out, sum_out, sq_out,
             acc_sum, acc_sq, *, nb):
    agg = p_ref[0] + p_ref[1]
    h = _mm(x_ref[...], w1a_ref[...]) + _mm(agg, w1b_ref[...]) + b1_ref[...]
    h_out[...] = h
    _accum_stats(h, sum_out, sq_out, acc_sum, acc_sq, nb)


def _l2_body(h_ref, sum_ref, sq_ref, g1_ref, be1_ref, w2_ref, b2_ref,
             h2_out, sum_out, sq_out, acc_sum, acc_sq, *, n, nb, eps):
    mean = sum_ref[...] * (1.0 / n)
    var = sq_ref[...] * (1.0 / n) - mean * mean
    h = (h_ref[...] - mean) * lax.rsqrt(var + eps) * g1_ref[...] + be1_ref[...]
    h = jnp.maximum(h, 0.0)
    h2 = _mm(h, w2_ref[...]) + b2_ref[...]
    h2_out[...] = h2
    _accum_stats(h2, sum_out, sq_out, acc_sum, acc_sq, nb)


def _l3_body(h2_ref, sum_ref, sq_ref, g2_ref, be2_ref, o_ref, *, n, eps):
    mean = sum_ref[...] * (1.0 / n)
    var = sq_ref[...] * (1.0 / n) - mean * mean
    h2 = (h2_ref[...] - mean) * lax.rsqrt(var + eps) * g2_ref[...] \
        + be2_ref[...]
    o_ref[...] = jnp.maximum(h2, 0.0)


def _mlp(x, partials, W1, b1, g1, be1, W2, b2, g2, be2, eps):
    n, d = x.shape
    d_hid = W1.shape[0]
    d_out = W2.shape[0]
    bl = 1000
    nb = n // bl
    w1a = W1[:, :d]
    w1b = W1[:, d:]
    row = lambda j: (j, 0)
    fixed = lambda j: (0, 0)
    stat_spec = lambda w: pl.BlockSpec((1, w), fixed)
    vec_spec = lambda w: pl.BlockSpec((w,), lambda j: (0,))

    h1pre, sum1, sq1 = pl.pallas_call(
        functools.partial(_l1_body, nb=nb),
        grid=(nb,),
        in_specs=[
            pl.BlockSpec((bl, d), row),
            pl.BlockSpec((2, bl, d), lambda j: (0, j, 0)),
            pl.BlockSpec((d_hid, d), fixed),
            pl.BlockSpec((d_hid, d), fixed),
            vec_spec(d_hid),
        ],
        out_specs=[pl.BlockSpec((bl, d_hid), row), stat_spec(d_hid),
                   stat_spec(d_hid)],
        out_shape=[jax.ShapeDtypeStruct((n, d_hid), jnp.float32),
                   jax.ShapeDtypeStruct((1, d_hid), jnp.float32),
                   jax.ShapeDtypeStruct((1, d_hid), jnp.float32)],
        scratch_shapes=[pltpu.VMEM((1, d_hid), jnp.float32),
                        pltpu.VMEM((1, d_hid), jnp.float32)],
    )(x, partials, w1a, w1b, b1)

    h2pre, sum2, sq2 = pl.pallas_call(
        functools.partial(_l2_body, n=n, nb=nb, eps=eps),
        grid=(nb,),
        in_specs=[
            pl.BlockSpec((bl, d_hid), row),
            stat_spec(d_hid),
            stat_spec(d_hid),
            vec_spec(d_hid),
            vec_spec(d_hid),
            pl.BlockSpec((d_out, d_hid), fixed),
            vec_spec(d_out),
        ],
        out_specs=[pl.BlockSpec((bl, d_out), row), stat_spec(d_out),
                   stat_spec(d_out)],
        out_shape=[jax.ShapeDtypeStruct((n, d_out), jnp.float32),
                   jax.ShapeDtypeStruct((1, d_out), jnp.float32),
                   jax.ShapeDtypeStruct((1, d_out), jnp.float32)],
        scratch_shapes=[pltpu.VMEM((1, d_out), jnp.float32),
                        pltpu.VMEM((1, d_out), jnp.float32)],
    )(h1pre, sum1, sq1, g1, be1, W2, b2)

    out = pl.pallas_call(
        functools.partial(_l3_body, n=n, eps=eps),
        grid=(nb,),
        in_specs=[
            pl.BlockSpec((bl, d_out), row),
            stat_spec(d_out),
            stat_spec(d_out),
            vec_spec(d_out),
            vec_spec(d_out),
        ],
        out_specs=pl.BlockSpec((bl, d_out), row),
        out_shape=jax.ShapeDtypeStruct((n, d_out), jnp.float32),
    )(h2pre, sum2, sq2, g2, be2)
    return out


def kernel(x, edge_index, W1, b1, g1, be1, W2, b2, g2, be2):
    n, d = x.shape
    e = edge_index.shape[1]
    eps = 1e-5

    src = edge_index[0].astype(jnp.int32)
    dst = edge_index[1].astype(jnp.int32)
    nw = NC * NS
    cpw = -(-e // (nw * CHUNK))
    cpw = -(-cpw // 4) * 4
    e_pad = nw * cpw * CHUNK
    n_pad = -(-(n + 1) // (NS * 8)) * (NS * 8)
    pad = e_pad - e
    pad_i = jnp.arange(pad, dtype=jnp.int32)
    src = jnp.concatenate([src, pad_i % n])
    dst = jnp.concatenate([dst, n + pad_i % (n_pad - n)])
    src3 = src.reshape(nw, cpw, CHUNK)
    dst3 = dst.reshape(nw, cpw, CHUNK)
    zeros_hbm = jnp.zeros((n_pad, d), jnp.float32)

    partials = _sc_segment_sum(x, src3, dst3, zeros_hbm, n_pad, cpw)
    return _mlp(x, partials, W1, b1, g1, be1, W2, b2, g2, be2, eps)


# speedup vs baseline: 3.0904x; 1.0490x over previous
import functools

import jax
import jax.numpy as jnp
from jax import lax
from jax.experimental import pallas as pl
from jax.experimental.pallas import tpu as pltpu
from jax.experimental.pallas import tpu_sc as plsc

NC = 2
NS = 16
CHUNK = 128


def _sc_segment_sum(x, src3, dst3, zeros_hbm, n_pad, cpw):
    d = x.shape[1]
    rows_per_tile = n_pad // NS

    half = cpw // 2

    def full_body(x_hbm, src_hbm, dst_hbm, z_hbm, out_hbm, src_v, dst_v,
                  buf_a, buf_b, acc, sem_a, sem_b):
        c = lax.axis_index("c")
        s = lax.axis_index("s")
        wid = s * NC + c
        bufs = (buf_a, buf_b)
        sems = (sem_a, sem_b)

        for h in range(2):
            pltpu.sync_copy(src_hbm.at[wid, pl.ds(h * half, half)], src_v)
            pltpu.sync_copy(dst_hbm.at[wid, pl.ds(h * half, half)], dst_v)

            if h == 0:
                @pl.when(s == 0)
                def _():
                    pltpu.sync_copy(z_hbm, acc)

                plsc.subcore_barrier()

            for b in range(2):
                pltpu.async_copy(x_hbm.at[src_v.at[b]], bufs[b], sems[b])

            def ring(t, carry):
                g = 2 * t
                for b in range(2):
                    j = g + b
                    pltpu.make_async_copy(x_hbm.at[src_v.at[j]], bufs[b],
                                          sems[b]).wait()
                    pltpu.sync_copy(bufs[b], acc.at[dst_v.at[j]], add=True)
                    pltpu.async_copy(x_hbm.at[src_v.at[j + 2]], bufs[b],
                                     sems[b])
                return carry

            lax.fori_loop(0, (half - 2) // 2, ring, 0)

            for b in range(2):
                j = half - 2 + b
                pltpu.make_async_copy(x_hbm.at[src_v.at[j]], bufs[b],
                                      sems[b]).wait()
                pltpu.sync_copy(bufs[b], acc.at[dst_v.at[j]], add=True)

        plsc.subcore_barrier()

        r0 = s * rows_per_tile
        pltpu.sync_copy(acc.at[pl.ds(r0, rows_per_tile)],
                        out_hbm.at[c, pl.ds(r0, rows_per_tile)])

    mesh = plsc.VectorSubcoreMesh(core_axis_name="c", subcore_axis_name="s")
    k = pl.kernel(
        full_body,
        out_type=jax.ShapeDtypeStruct((NC, n_pad, d), jnp.float32),
        mesh=mesh,
        scratch_types=[
            pltpu.VMEM((half, CHUNK), jnp.int32),
            pltpu.VMEM((half, CHUNK), jnp.int32),
            pltpu.VMEM((CHUNK, d), jnp.float32),
            pltpu.VMEM((CHUNK, d), jnp.float32),
            pltpu.VMEM_SHARED((n_pad, d), jnp.float32),
            pltpu.SemaphoreType.DMA,
            pltpu.SemaphoreType.DMA,
        ],
    )
    return k(x, src3, dst3, zeros_hbm)


def _mm(a, b_t):
    return lax.dot_general(a, b_t, (((1,), (1,)), ((), ())),
                           preferred_element_type=jnp.float32,
                           precision=lax.Precision.HIGHEST)


def _fused_mlp_body(x_ref, p_ref, w1a_ref, w1b_ref, b1_ref, g1_ref, be1_ref,
                    w2_ref, b2_ref, g2_ref, be2_ref, o_ref,
                    h1_scr, h2_scr, sum1, sq1, sum2, sq2, *, n, bl, nb, eps):
    p = pl.program_id(0)
    j = pl.program_id(1)
    rows = pl.ds(j * bl, bl)

    @pl.when(p == 0)
    def _():
        agg = p_ref[0] + p_ref[1]
        h = (_mm(x_ref[...], w1a_ref[...]) + _mm(agg, w1b_ref[...])
             + b1_ref[...])
        h1_scr[rows, :] = h
        s = jnp.sum(h, axis=0, keepdims=True)
        q = jnp.sum(h * h, axis=0, keepdims=True)

        @pl.when(j == 0)
        def _():
            sum1[...] = s
            sq1[...] = q

        @pl.when(j > 0)
        def _():
            sum1[...] += s
            sq1[...] += q

    @pl.when(p == 1)
    def _():
        mean = sum1[...] * (1.0 / n)
        var = sq1[...] * (1.0 / n) - mean * mean
        h = ((h1_scr[rows, :] - mean) * lax.rsqrt(var + eps) * g1_ref[...]
             + be1_ref[...])
        h = jnp.maximum(h, 0.0)
        h2 = _mm(h, w2_ref[...]) + b2_ref[...]
        h2_scr[rows, :] = h2
        s = jnp.sum(h2, axis=0, keepdims=True)
        q = jnp.sum(h2 * h2, axis=0, keepdims=True)

        @pl.when(j == 0)
        def _():
            sum2[...] = s
            sq2[...] = q

        @pl.when(j > 0)
        def _():
            sum2[...] += s
            sq2[...] += q

    @pl.when(p == 2)
    def _():
        mean = sum2[...] * (1.0 / n)
        var = sq2[...] * (1.0 / n) - mean * mean
        h2 = ((h2_scr[rows, :] - mean) * lax.rsqrt(var + eps) * g2_ref[...]
              + be2_ref[...])
        o_ref[...] = jnp.maximum(h2, 0.0)


def _mlp(x, partials, W1, b1, g1, be1, W2, b2, g2, be2, eps):
    n, d = x.shape
    d_hid = W1.shape[0]
    d_out = W2.shape[0]
    bl = 2000
    nb = n // bl
    w1a = W1[:, :d]
    w1b = W1[:, d:]
    row_if = lambda ph: (lambda p, j: (jnp.where(p == ph, j, 0), 0))
    fixed = lambda p, j: (0, 0)
    vec_spec = lambda w: pl.BlockSpec((w,), lambda p, j: (0,))

    return pl.pallas_call(
        functools.partial(_fused_mlp_body, n=n, bl=bl, nb=nb, eps=eps),
        grid=(3, nb),
        in_specs=[
            pl.BlockSpec((bl, d), row_if(0)),
            pl.BlockSpec((2, bl, d), lambda p, j: (0, jnp.where(p == 0, j, 0),
                                                   0)),
            pl.BlockSpec((d_hid, d), fixed),
            pl.BlockSpec((d_hid, d), fixed),
            vec_spec(d_hid),
            vec_spec(d_hid),
            vec_spec(d_hid),
            pl.BlockSpec((d_out, d_hid), fixed),
            vec_spec(d_out),
            vec_spec(d_out),
            vec_spec(d_out),
        ],
        out_specs=pl.BlockSpec((bl, d_out), row_if(2)),
        out_shape=jax.ShapeDtypeStruct((n, d_out), jnp.float32),
        scratch_shapes=[
            pltpu.VMEM((n, d_hid), jnp.float32),
            pltpu.VMEM((n, d_out), jnp.float32),
            pltpu.VMEM((1, d_hid), jnp.float32),
            pltpu.VMEM((1, d_hid), jnp.float32),
            pltpu.VMEM((1, d_out), jnp.float32),
            pltpu.VMEM((1, d_out), jnp.float32),
        ],
        compiler_params=pltpu.CompilerParams(
            vmem_limit_bytes=60 * 1024 * 1024),
    )(x, partials, w1a, w1b, b1, g1, be1, W2, b2, g2, be2)


def kernel(x, edge_index, W1, b1, g1, be1, W2, b2, g2, be2):
    n, d = x.shape
    e = edge_index.shape[1]
    eps = 1e-5

    src = edge_index[0].astype(jnp.int32)
    dst = edge_index[1].astype(jnp.int32)
    nw = NC * NS
    cpw = -(-e // (nw * CHUNK))
    cpw = -(-cpw // 4) * 4
    e_pad = nw * cpw * CHUNK
    n_pad = -(-(n + 1) // (NS * 8)) * (NS * 8)
    pad = e_pad - e
    pad_i = jnp.arange(pad, dtype=jnp.int32)
    src = jnp.concatenate([src, pad_i % n])
    dst = jnp.concatenate([dst, n + pad_i % (n_pad - n)])
    src3 = src.reshape(nw, cpw, CHUNK)
    dst3 = dst.reshape(nw, cpw, CHUNK)
    zeros_hbm = jnp.zeros((n_pad, d), jnp.float32)

    partials = _sc_segment_sum(x, src3, dst3, zeros_hbm, n_pad, cpw)
    return _mlp(x, partials, W1, b1, g1, be1, W2, b2, g2, be2, eps)


# speedup vs baseline: 3.8998x; 1.2619x over previous
import functools

import jax
import jax.numpy as jnp
from jax import lax
from jax.experimental import pallas as pl
from jax.experimental.pallas import tpu as pltpu
from jax.experimental.pallas import tpu_sc as plsc

NC = 2
NS = 16
CHUNK = 128


def _sc_segment_sum(x, src3, dst3, zeros_hbm, n_pad, cpw):
    d = x.shape[1]
    rows_per_tile = n_pad // NS

    half = cpw // 2

    def full_body(x_hbm, src_hbm, dst_hbm, z_hbm, out_hbm, src_v, dst_v,
                  buf_a, buf_b, acc, sem_a, sem_b):
        c = lax.axis_index("c")
        s = lax.axis_index("s")
        wid = s * NC + c
        bufs = (buf_a, buf_b)
        sems = (sem_a, sem_b)

        for h in range(2):
            pltpu.sync_copy(src_hbm.at[wid, pl.ds(h * half, half)], src_v)
            pltpu.sync_copy(dst_hbm.at[wid, pl.ds(h * half, half)], dst_v)

            if h == 0:
                @pl.when(s == 0)
                def _():
                    pltpu.sync_copy(z_hbm, acc)

                plsc.subcore_barrier()

            for b in range(2):
                pltpu.async_copy(x_hbm.at[src_v.at[b]], bufs[b], sems[b])

            def ring(t, carry):
                g = 2 * t
                for b in range(2):
                    j = g + b
                    pltpu.make_async_copy(x_hbm.at[src_v.at[j]], bufs[b],
                                          sems[b]).wait()
                    pltpu.sync_copy(bufs[b], acc.at[dst_v.at[j]], add=True)
                    pltpu.async_copy(x_hbm.at[src_v.at[j + 2]], bufs[b],
                                     sems[b])
                return carry

            lax.fori_loop(0, (half - 2) // 2, ring, 0)

            for b in range(2):
                j = half - 2 + b
                pltpu.make_async_copy(x_hbm.at[src_v.at[j]], bufs[b],
                                      sems[b]).wait()
                pltpu.sync_copy(bufs[b], acc.at[dst_v.at[j]], add=True)

        plsc.subcore_barrier()

        r0 = s * rows_per_tile
        pltpu.sync_copy(acc.at[pl.ds(r0, rows_per_tile)],
                        out_hbm.at[c, pl.ds(r0, rows_per_tile)])

    mesh = plsc.VectorSubcoreMesh(core_axis_name="c", subcore_axis_name="s")
    k = pl.kernel(
        full_body,
        out_type=jax.ShapeDtypeStruct((NC, n_pad, d), jnp.float32),
        mesh=mesh,
        scratch_types=[
            pltpu.VMEM((half, CHUNK), jnp.int32),
            pltpu.VMEM((half, CHUNK), jnp.int32),
            pltpu.VMEM((CHUNK, d), jnp.float32),
            pltpu.VMEM((CHUNK, d), jnp.float32),
            pltpu.VMEM_SHARED((n_pad, d), jnp.float32),
            pltpu.SemaphoreType.DMA,
            pltpu.SemaphoreType.DMA,
        ],
    )
    return k(x, src3, dst3, zeros_hbm)


def _mm(a, b_t):
    return lax.dot_general(a, b_t, (((1,), (1,)), ((), ())),
                           preferred_element_type=jnp.float32,
                           precision=lax.Precision.DEFAULT)


def _fused_mlp_body(x_ref, p_ref, w1a_ref, w1b_ref, b1_ref, g1_ref, be1_ref,
                    w2_ref, b2_ref, g2_ref, be2_ref, o_ref,
                    h1_scr, h2_scr, sum1, sq1, sum2, sq2, *, n, bl, nb, eps):
    p = pl.program_id(0)
    j = pl.program_id(1)
    rows = pl.ds(j * bl, bl)

    @pl.when(p == 0)
    def _():
        agg = p_ref[0] + p_ref[1]
        h = (_mm(x_ref[...], w1a_ref[...]) + _mm(agg, w1b_ref[...])
             + b1_ref[...])
        h1_scr[rows, :] = h
        s = jnp.sum(h, axis=0, keepdims=True)
        q = jnp.sum(h * h, axis=0, keepdims=True)

        @pl.when(j == 0)
        def _():
            sum1[...] = s
            sq1[...] = q

        @pl.when(j > 0)
        def _():
            sum1[...] += s
            sq1[...] += q

    @pl.when(p == 1)
    def _():
        mean = sum1[...] * (1.0 / n)
        var = sq1[...] * (1.0 / n) - mean * mean
        h = ((h1_scr[rows, :] - mean) * lax.rsqrt(var + eps) * g1_ref[...]
             + be1_ref[...])
        h = jnp.maximum(h, 0.0)
        h2 = _mm(h, w2_ref[...]) + b2_ref[...]
        h2_scr[rows, :] = h2
        s = jnp.sum(h2, axis=0, keepdims=True)
        q = jnp.sum(h2 * h2, axis=0, keepdims=True)

        @pl.when(j == 0)
        def _():
            sum2[...] = s
            sq2[...] = q

        @pl.when(j > 0)
        def _():
            sum2[...] += s
            sq2[...] += q

    @pl.when(p == 2)
    def _():
        mean = sum2[...] * (1.0 / n)
        var = sq2[...] * (1.0 / n) - mean * mean
        h2 = ((h2_scr[rows, :] - mean) * lax.rsqrt(var + eps) * g2_ref[...]
              + be2_ref[...])
        o_ref[...] = jnp.maximum(h2, 0.0)


def _mlp(x, partials, W1, b1, g1, be1, W2, b2, g2, be2, eps):
    n, d = x.shape
    d_hid = W1.shape[0]
    d_out = W2.shape[0]
    bl = 2000
    nb = n // bl
    w1a = W1[:, :d]
    w1b = W1[:, d:]
    row_if = lambda ph: (lambda p, j: (jnp.where(p == ph, j, 0), 0))
    fixed = lambda p, j: (0, 0)
    vec_spec = lambda w: pl.BlockSpec((w,), lambda p, j: (0,))

    return pl.pallas_call(
        functools.partial(_fused_mlp_body, n=n, bl=bl, nb=nb, eps=eps),
        grid=(3, nb),
        in_specs=[
            pl.BlockSpec((bl, d), row_if(0)),
            pl.BlockSpec((2, bl, d), lambda p, j: (0, jnp.where(p == 0, j, 0),
                                                   0)),
            pl.BlockSpec((d_hid, d), fixed),
            pl.BlockSpec((d_hid, d), fixed),
            vec_spec(d_hid),
            vec_spec(d_hid),
            vec_spec(d_hid),
            pl.BlockSpec((d_out, d_hid), fixed),
            vec_spec(d_out),
            vec_spec(d_out),
            vec_spec(d_out),
        ],
        out_specs=pl.BlockSpec((bl, d_out), row_if(2)),
        out_shape=jax.ShapeDtypeStruct((n, d_out), jnp.float32),
        scratch_shapes=[
            pltpu.VMEM((n, d_hid), jnp.float32),
            pltpu.VMEM((n, d_out), jnp.float32),
            pltpu.VMEM((1, d_hid), jnp.float32),
            pltpu.VMEM((1, d_hid), jnp.float32),
            pltpu.VMEM((1, d_out), jnp.float32),
            pltpu.VMEM((1, d_out), jnp.float32),
        ],
        compiler_params=pltpu.CompilerParams(
            vmem_limit_bytes=60 * 1024 * 1024),
    )(x, partials, w1a, w1b, b1, g1, be1, W2, b2, g2, be2)


def kernel(x, edge_index, W1, b1, g1, be1, W2, b2, g2, be2):
    n, d = x.shape
    e = edge_index.shape[1]
    eps = 1e-5

    src = edge_index[0].astype(jnp.int32)
    dst = edge_index[1].astype(jnp.int32)
    nw = NC * NS
    cpw = -(-e // (nw * CHUNK))
    cpw = -(-cpw // 4) * 4
    e_pad = nw * cpw * CHUNK
    n_pad = -(-(n + 1) // (NS * 8)) * (NS * 8)
    pad = e_pad - e
    pad_i = jnp.arange(pad, dtype=jnp.int32)
    src = jnp.concatenate([src, pad_i % n])
    dst = jnp.concatenate([dst, n + pad_i % (n_pad - n)])
    src3 = src.reshape(nw, cpw, CHUNK)
    dst3 = dst.reshape(nw, cpw, CHUNK)
    zeros_hbm = jnp.zeros((n_pad, d), jnp.float32)

    partials = _sc_segment_sum(x, src3, dst3, zeros_hbm, n_pad, cpw)
    return _mlp(x, partials, W1, b1, g1, be1, W2, b2, g2, be2, eps)
